# Initial kernel scaffold; baseline (speedup 1.0000x reference)
import jax
import jax.numpy as jnp
from jax.experimental import pallas as pl


def kernel(x, pos_table, hash_index):
    raise NotImplementedError("write your pallas kernel here")


# speedup vs baseline: 1.0651x; 1.0651x over previous
import functools

import jax
import jax.numpy as jnp
from jax import lax
from jax.experimental import pallas as pl
from jax.experimental.pallas import tpu as pltpu
from jax.experimental.pallas import tpu_sc as plsc

D_HID = 1024
N_POS = 4096
TRAIN_H = 64
TRAIN_W = 64
N_SEL = TRAIN_H * TRAIN_W

_NUM_CORES = 2
_NUM_SUBCORES = 16
_NW = _NUM_CORES * _NUM_SUBCORES
_ROWS_PER_W = N_SEL // _NW
_CHUNK = 32
_NCHUNK = _ROWS_PER_W // _CHUNK


def _sc_gather(table, idx3):
    mesh = plsc.VectorSubcoreMesh(core_axis_name="c", subcore_axis_name="s")

    @functools.partial(
        pl.kernel,
        out_type=jax.ShapeDtypeStruct((N_SEL, D_HID), jnp.float32),
        mesh=mesh,
        scratch_types=[
            pltpu.VMEM((_NCHUNK, _CHUNK), jnp.int32),
            pltpu.VMEM((_CHUNK, D_HID), jnp.float32),
            pltpu.VMEM((_CHUNK, D_HID), jnp.float32),
            pltpu.SemaphoreType.DMA,
            pltpu.SemaphoreType.DMA,
        ],
    )
    def gather_kernel(table_hbm, idx_hbm, out_hbm, idx_v, buf0, buf1, sem0, sem1):
        wid = lax.axis_index("s") * _NUM_CORES + lax.axis_index("c")
        base = wid * _ROWS_PER_W
        pltpu.sync_copy(idx_hbm.at[wid], idx_v)
        bufs = (buf0, buf1)
        sems = (sem0, sem1)
        copies = [None] * _NCHUNK
        copies[0] = pltpu.async_copy(
            table_hbm.at[idx_v.at[0]], bufs[0], sems[0])
        for k in range(_NCHUNK):
            if k + 1 < _NCHUNK:
                copies[k + 1] = pltpu.async_copy(
                    table_hbm.at[idx_v.at[k + 1]],
                    bufs[(k + 1) % 2], sems[(k + 1) % 2])
            copies[k].wait()
            pltpu.sync_copy(bufs[k % 2],
                            out_hbm.at[pl.ds(base + k * _CHUNK, _CHUNK)])

    return gather_kernel(table, idx3)


_ROW_BLK = 512


def _tc_add_body(x_ref, pe_ref, o_ref):
    o_ref[...] = x_ref[...] + pe_ref[...][None, :, :]


def _tc_add(x, pe):
    b, n, d = x.shape
    nrb = n // _ROW_BLK
    return pl.pallas_call(
        _tc_add_body,
        grid=(nrb, b),
        in_specs=[
            pl.BlockSpec((1, _ROW_BLK, d), lambda r, bb: (bb, r, 0)),
            pl.BlockSpec((_ROW_BLK, d), lambda r, bb: (r, 0)),
        ],
        out_specs=pl.BlockSpec((1, _ROW_BLK, d), lambda r, bb: (bb, r, 0)),
        out_shape=jax.ShapeDtypeStruct(x.shape, x.dtype),
    )(x, pe)


def kernel(x, pos_table, hash_index):
    sel = hash_index[:TRAIN_H, :TRAIN_W].reshape(-1).astype(jnp.int32)
    idx3 = sel.reshape(_NW, _NCHUNK, _CHUNK)
    table = pos_table.reshape(N_POS, D_HID)
    pe = _sc_gather(table, idx3)
    return _tc_add(x, pe)


# speedup vs baseline: 1.1495x; 1.0792x over previous
import functools

import jax
import jax.numpy as jnp
from jax import lax
from jax.experimental import pallas as pl
from jax.experimental.pallas import tpu as pltpu
from jax.experimental.pallas import tpu_sc as plsc

D_HID = 1024
N_POS = 4096
TRAIN_H = 64
TRAIN_W = 64
N_SEL = TRAIN_H * TRAIN_W

_NUM_CORES = 2
_NUM_SUBCORES = 16
_NW = _NUM_CORES * _NUM_SUBCORES
_ROWS_PER_W = N_SEL // _NW
_CHUNK = 32
_NCHUNK = _ROWS_PER_W // _CHUNK


def _sc_gather(table, idx3):
    mesh = plsc.VectorSubcoreMesh(core_axis_name="c", subcore_axis_name="s")

    @functools.partial(
        pl.kernel,
        out_type=jax.ShapeDtypeStruct((N_SEL, D_HID), jnp.float32),
        mesh=mesh,
        scratch_types=[
            pltpu.VMEM((_NCHUNK, _CHUNK), jnp.int32),
            pltpu.VMEM((_CHUNK, D_HID), jnp.float32),
            pltpu.VMEM((_CHUNK, D_HID), jnp.float32),
            pltpu.SemaphoreType.DMA,
            pltpu.SemaphoreType.DMA,
        ],
    )
    def gather_kernel(table_hbm, idx_hbm, out_hbm, idx_v, buf0, buf1, sem0, sem1):
        wid = lax.axis_index("s") * _NUM_CORES + lax.axis_index("c")
        base = wid * _ROWS_PER_W
        pltpu.sync_copy(idx_hbm.at[wid], idx_v)
        bufs = (buf0, buf1)
        sems = (sem0, sem1)
        copies = [None] * _NCHUNK
        copies[0] = pltpu.async_copy(
            table_hbm.at[idx_v.at[0]], bufs[0], sems[0])
        for k in range(_NCHUNK):
            if k + 1 < _NCHUNK:
                copies[k + 1] = pltpu.async_copy(
                    table_hbm.at[idx_v.at[k + 1]],
                    bufs[(k + 1) % 2], sems[(k + 1) % 2])
            copies[k].wait()
            pltpu.sync_copy(bufs[k % 2],
                            out_hbm.at[pl.ds(base + k * _CHUNK, _CHUNK)])

    return gather_kernel(table, idx3)


_ROW_BLK = 256


def _tc_add_body(x_ref, pe_ref, o_ref):
    o_ref[...] = x_ref[...] + pe_ref[...][None, :, :]


def _tc_add(x, pe):
    b, n, d = x.shape
    nrb = n // _ROW_BLK
    return pl.pallas_call(
        _tc_add_body,
        grid=(nrb,),
        in_specs=[
            pl.BlockSpec((b, _ROW_BLK, d), lambda r: (0, r, 0)),
            pl.BlockSpec((_ROW_BLK, d), lambda r: (r, 0)),
        ],
        out_specs=pl.BlockSpec((b, _ROW_BLK, d), lambda r: (0, r, 0)),
        out_shape=jax.ShapeDtypeStruct(x.shape, x.dtype),
    )(x, pe)


def kernel(x, pos_table, hash_index):
    sel = hash_index[:TRAIN_H, :TRAIN_W].reshape(-1).astype(jnp.int32)
    idx3 = sel.reshape(_NW, _NCHUNK, _CHUNK)
    table = pos_table.reshape(N_POS, D_HID)
    pe = _sc_gather(table, idx3)
    return _tc_add(x, pe)
